# baseline (speedup 1.0000x reference)
import functools

import jax
import jax.numpy as jnp
from jax import lax
from jax.experimental import pallas as pl
from jax.experimental.pallas import tpu as pltpu
from jax.experimental.pallas import tpu_sc as plsc

N = 10000
E = 160000
DF = 256
DC = 128
DH = 64

NC, NS = 2, 16
CHUNK = 125
ROWS = E // CHUNK
ROWS_L1 = ROWS // NS
ROWS_L2 = ROWS // (NC * NS)
PH = 5
PHROWS = ROWS_L1 // PH
WB = 80
NCH = N // WB
KWB = -(-NCH // NS)


def _lin1_body(x_ref, wrel_ref, wroot_ref, b_ref, ya_ref, yb_ref, za_ref, zb_ref):
    xb = x_ref[...]
    dn = (((1,), (1,)), ((), ()))
    y = lax.dot_general(xb, wrel_ref[...], dn, preferred_element_type=jnp.float32)
    z = lax.dot_general(xb, wroot_ref[...], dn, preferred_element_type=jnp.float32)
    z = z + b_ref[...]
    ya_ref[...] = y[:, :DC]
    yb_ref[...] = y[:, DC:]
    za_ref[...] = z[:, :DC]
    zb_ref[...] = z[:, DC:]


def _lin2_body(ga_ref, gb_ref, wrel_ref, wroot_ref, b_ref, y_ref, z_ref):
    h = jnp.maximum(jnp.concatenate([ga_ref[...], gb_ref[...]], axis=1), 0.0)
    h = h.astype(jnp.bfloat16)
    dn = (((1,), (1,)), ((), ()))
    y_ref[...] = lax.dot_general(h, wrel_ref[...], dn,
                                 preferred_element_type=jnp.float32)
    z = lax.dot_general(h, wroot_ref[...], dn, preferred_element_type=jnp.float32)
    z_ref[...] = (z + b_ref[...]) * 0.5


_BR = 2000


def _linear1(x, wrel, wroot, b):
    return pl.pallas_call(
        _lin1_body,
        grid=(N // _BR,),
        in_specs=[
            pl.BlockSpec((_BR, DF), lambda i: (i, 0)),
            pl.BlockSpec((DF, DF), lambda i: (0, 0)),
            pl.BlockSpec((DF, DF), lambda i: (0, 0)),
            pl.BlockSpec((1, DF), lambda i: (0, 0)),
        ],
        out_specs=[pl.BlockSpec((_BR, DC), lambda i: (i, 0))] * 4,
        out_shape=[jax.ShapeDtypeStruct((N, DC), jnp.float32)] * 4,
    )(x, wrel, wroot, b)


def _linear2(ga, gb, wrel, wroot, b):
    return pl.pallas_call(
        _lin2_body,
        grid=(N // _BR,),
        in_specs=[
            pl.BlockSpec((_BR, DC), lambda i: (i, 0)),
            pl.BlockSpec((_BR, DC), lambda i: (i, 0)),
            pl.BlockSpec((DC, DF), lambda i: (0, 0)),
            pl.BlockSpec((DC, DF), lambda i: (0, 0)),
            pl.BlockSpec((1, DC), lambda i: (0, 0)),
        ],
        out_specs=[pl.BlockSpec((_BR, DC), lambda i: (i, 0))] * 2,
        out_shape=[jax.ShapeDtypeStruct((N, DC), jnp.float32)] * 2,
    )(ga, gb, wrel, wroot, b)


def _comb_body(a_ref, b_ref, o_ref):
    o_ref[...] = a_ref[...] + b_ref[...]


def _combine(pa, pb):
    return pl.pallas_call(
        _comb_body,
        grid=(N // _BR,),
        in_specs=[pl.BlockSpec((_BR, DC), lambda i: (i, 0))] * 2,
        out_specs=pl.BlockSpec((_BR, DC), lambda i: (i, 0)),
        out_shape=jax.ShapeDtypeStruct((N, DC), jnp.float32),
    )(pa, pb)


@functools.cache
def _mesh():
    return plsc.VectorSubcoreMesh(
        core_axis_name="c", subcore_axis_name="s", num_cores=NC, num_subcores=NS)


def _init_acc(s, z_hbm, acc, sem):
    def body(k, _):
        ci = k * NS + s
        @pl.when(ci < NCH)
        def _():
            r0 = ci * WB
            pltpu.async_copy(z_hbm.at[pl.ds(r0, WB)], acc.at[pl.ds(r0, WB)], sem)
        return 0
    lax.fori_loop(0, KWB, body, 0)
    def drain(k, _):
        ci = k * NS + s
        @pl.when(ci < NCH)
        def _():
            pltpu.make_async_copy(z_hbm.at[pl.ds(0, WB)], acc.at[pl.ds(0, WB)], sem).wait()
        return 0
    lax.fori_loop(0, KWB, drain, 0)


def _writeback(s, acc, out_hbm, sem):
    def body(k, _):
        ci = k * NS + s
        @pl.when(ci < NCH)
        def _():
            r0 = ci * WB
            pltpu.async_copy(acc.at[pl.ds(r0, WB)], out_hbm.at[pl.ds(r0, WB)], sem)
        return 0
    lax.fori_loop(0, KWB, body, 0)
    def drain(k, _):
        ci = k * NS + s
        @pl.when(ci < NCH)
        def _():
            pltpu.make_async_copy(acc.at[pl.ds(0, WB)], out_hbm.at[pl.ds(0, WB)], sem).wait()
        return 0
    lax.fori_loop(0, KWB, drain, 0)


def _edge_loop(n_rows, y_hbm, src_v, dst_v, acc, b0, b1, gs0, gs1, ss0, ss1):
    def gather(i, buf, sem):
        pltpu.async_copy(y_hbm.at[src_v.at[i]], buf, sem)

    def scat(i, buf, sem):
        pltpu.async_copy(buf, acc.at[dst_v.at[i]], sem, add=True)

    def wait_g(buf, sem):
        pltpu.make_async_copy(y_hbm.at[src_v.at[0]], buf, sem).wait()

    def wait_s(buf, sem):
        pltpu.make_async_copy(buf, acc.at[dst_v.at[0]], sem).wait()

    gather(0, b0, gs0)

    def body(k, _):
        i0 = 2 * k
        pl.when(k > 0)(lambda: wait_s(b1, ss1))
        gather(i0 + 1, b1, gs1)
        wait_g(b0, gs0)
        scat(i0, b0, ss0)

        @pl.when(i0 + 2 < n_rows)
        def _():
            wait_s(b0, ss0)
            gather(i0 + 2, b0, gs0)

        wait_g(b1, gs1)
        scat(i0 + 1, b1, ss1)
        return 0

    lax.fori_loop(0, n_rows // 2, body, 0)
    wait_s(b0, ss0)
    wait_s(b1, ss1)


def _idx_load(src_hbm, dst_hbm, r0, sbuf, dbuf, sem):
    pltpu.async_copy(src_hbm.at[pl.ds(r0, PHROWS)], sbuf, sem)
    pltpu.async_copy(dst_hbm.at[pl.ds(r0, PHROWS)], dbuf, sem)


def _idx_wait(src_hbm, dst_hbm, sbuf, dbuf, sem):
    pltpu.make_async_copy(src_hbm.at[pl.ds(0, PHROWS)], sbuf, sem).wait()
    pltpu.make_async_copy(dst_hbm.at[pl.ds(0, PHROWS)], dbuf, sem).wait()


@functools.cache
def _sc_layer1():
    @functools.partial(
        pl.kernel,
        out_type=(jax.ShapeDtypeStruct((N, DC), jnp.float32),) * 2,
        mesh=_mesh(),
        scratch_types=[
            pltpu.VMEM((PHROWS, CHUNK), jnp.int32),
            pltpu.VMEM((PHROWS, CHUNK), jnp.int32),
            pltpu.VMEM((PHROWS, CHUNK), jnp.int32),
            pltpu.VMEM((PHROWS, CHUNK), jnp.int32),
            pltpu.VMEM((CHUNK, DC), jnp.float32),
            pltpu.VMEM((CHUNK, DC), jnp.float32),
            pltpu.VMEM_SHARED((N, DC), jnp.float32),
            pltpu.SemaphoreType.DMA, pltpu.SemaphoreType.DMA,
            pltpu.SemaphoreType.DMA, pltpu.SemaphoreType.DMA,
            pltpu.SemaphoreType.DMA, pltpu.SemaphoreType.DMA,
        ],
    )
    def sc1(src_hbm, dst_hbm, ya_hbm, yb_hbm, za_hbm, zb_hbm,
            ga_hbm, gb_hbm, sv0, dv0, sv1, dv1, b0, b1, acc,
            gs0, gs1, ss0, ss1, is0, is1):
        c = lax.axis_index("c")
        s = lax.axis_index("s")
        idx = [(sv0, dv0, is0), (sv1, dv1, is1)]

        def run(y_hbm, z_hbm, out_hbm):
            _idx_load(src_hbm, dst_hbm, s * ROWS_L1, sv0, dv0, is0)
            _init_acc(s, z_hbm, acc, gs0)
            plsc.subcore_barrier()
            for p in range(PH):
                sb, db, sem = idx[p % 2]
                if p + 1 < PH:
                    nb, nd, nsem = idx[(p + 1) % 2]
                    _idx_load(src_hbm, dst_hbm,
                              s * ROWS_L1 + (p + 1) * PHROWS, nb, nd, nsem)
                _idx_wait(src_hbm, dst_hbm, sb, db, sem)
                _edge_loop(PHROWS, y_hbm, sb, db, acc, b0, b1, gs0, gs1, ss0, ss1)
            plsc.subcore_barrier()
            _writeback(s, acc, out_hbm, gs0)

        pl.when(c == 0)(lambda: run(ya_hbm, za_hbm, ga_hbm))
        pl.when(c == 1)(lambda: run(yb_hbm, zb_hbm, gb_hbm))

    return sc1


@functools.cache
def _sc_layer2():
    @functools.partial(
        pl.kernel,
        out_type=(jax.ShapeDtypeStruct((N, DC), jnp.float32),) * 2,
        mesh=_mesh(),
        scratch_types=[
            pltpu.VMEM((ROWS_L2, CHUNK), jnp.int32),
            pltpu.VMEM((ROWS_L2, CHUNK), jnp.int32),
            pltpu.VMEM((CHUNK, DC), jnp.float32),
            pltpu.VMEM((CHUNK, DC), jnp.float32),
            pltpu.VMEM_SHARED((N, DC), jnp.float32),
            pltpu.SemaphoreType.DMA, pltpu.SemaphoreType.DMA,
            pltpu.SemaphoreType.DMA, pltpu.SemaphoreType.DMA,
            pltpu.SemaphoreType.DMA,
        ],
    )
    def sc2(src_hbm, dst_hbm, y_hbm, z_hbm,
            oa_hbm, ob_hbm, src_v, dst_v, b0, b1, acc, gs0, gs1, ss0, ss1, is0):
        c = lax.axis_index("c")
        s = lax.axis_index("s")
        r0 = (c * NS + s) * ROWS_L2
        pltpu.async_copy(src_hbm.at[pl.ds(r0, ROWS_L2)], src_v, is0)
        pltpu.async_copy(dst_hbm.at[pl.ds(r0, ROWS_L2)], dst_v, is0)
        _init_acc(s, z_hbm, acc, gs0)
        pltpu.make_async_copy(src_hbm.at[pl.ds(0, ROWS_L2)], src_v, is0).wait()
        pltpu.make_async_copy(dst_hbm.at[pl.ds(0, ROWS_L2)], dst_v, is0).wait()
        plsc.subcore_barrier()
        _edge_loop(ROWS_L2, y_hbm, src_v, dst_v, acc, b0, b1, gs0, gs1, ss0, ss1)
        plsc.subcore_barrier()
        pl.when(c == 0)(lambda: _writeback(s, acc, oa_hbm, gs0))
        pl.when(c == 1)(lambda: _writeback(s, acc, ob_hbm, gs0))

    return sc2


def kernel(edge_index, x, W1_rel, W1_root, b1, W2_rel, W2_root, b2):
    src2d = edge_index[0].reshape(ROWS, CHUNK)
    dst2d = edge_index[1].reshape(ROWS, CHUNK)
    bf16 = jnp.bfloat16
    y1a, y1b, z1a, z1b = _linear1(
        x.astype(bf16), W1_rel.astype(bf16), W1_root.astype(bf16),
        b1.reshape(1, -1))
    g1a, g1b = _sc_layer1()(src2d, dst2d, y1a, y1b, z1a, z1b)
    y2, z2h = _linear2(g1a, g1b, W2_rel.astype(bf16), W2_root.astype(bf16),
                       b2.reshape(1, -1))
    pa, pb = _sc_layer2()(src2d, dst2d, y2, z2h)
    return _combine(pa, pb)


# speedup vs baseline: 1.0496x; 1.0496x over previous
import functools

import jax
import jax.numpy as jnp
from jax import lax
from jax.experimental import pallas as pl
from jax.experimental.pallas import tpu as pltpu
from jax.experimental.pallas import tpu_sc as plsc

N = 10000
E = 160000
DF = 256
DC = 128
DH = 64

NC, NS = 2, 16
CHUNK = 125
ROWS = E // CHUNK
ROWS_L1 = ROWS // NS
ROWS_L2 = ROWS // (NC * NS)
PH = 2
PHROWS = ROWS_L1 // PH
WB = 80
NCH = N // WB
KWB = -(-NCH // NS)


def _lin1_body(x_ref, wrel_ref, wroot_ref, b_ref, ya_ref, yb_ref, za_ref, zb_ref):
    xb = x_ref[...]
    dn = (((1,), (1,)), ((), ()))
    y = lax.dot_general(xb, wrel_ref[...], dn, preferred_element_type=jnp.float32)
    z = lax.dot_general(xb, wroot_ref[...], dn, preferred_element_type=jnp.float32)
    z = z + b_ref[...]
    ya_ref[...] = y[:, :DC]
    yb_ref[...] = y[:, DC:]
    za_ref[...] = z[:, :DC]
    zb_ref[...] = z[:, DC:]


def _lin2_body(ga_ref, gb_ref, wrel_ref, wroot_ref, b_ref, y_ref, z_ref):
    h = jnp.maximum(jnp.concatenate([ga_ref[...], gb_ref[...]], axis=1), 0.0)
    dn = (((1,), (1,)), ((), ()))
    y_ref[...] = lax.dot_general(h, wrel_ref[...], dn,
                                 preferred_element_type=jnp.float32)
    z = lax.dot_general(h, wroot_ref[...], dn, preferred_element_type=jnp.float32)
    z_ref[...] = (z + b_ref[...]) * 0.5


_BR = 5000


def _linear1(x, wrel, wroot, b):
    return pl.pallas_call(
        _lin1_body,
        grid=(N // _BR,),
        in_specs=[
            pl.BlockSpec((_BR, DF), lambda i: (i, 0)),
            pl.BlockSpec((DF, DF), lambda i: (0, 0)),
            pl.BlockSpec((DF, DF), lambda i: (0, 0)),
            pl.BlockSpec((1, DF), lambda i: (0, 0)),
        ],
        out_specs=[pl.BlockSpec((_BR, DC), lambda i: (i, 0))] * 4,
        out_shape=[jax.ShapeDtypeStruct((N, DC), jnp.float32)] * 4,
    )(x, wrel, wroot, b)


def _linear2(ga, gb, wrel, wroot, b):
    return pl.pallas_call(
        _lin2_body,
        grid=(N // _BR,),
        in_specs=[
            pl.BlockSpec((_BR, DC), lambda i: (i, 0)),
            pl.BlockSpec((_BR, DC), lambda i: (i, 0)),
            pl.BlockSpec((DC, DF), lambda i: (0, 0)),
            pl.BlockSpec((DC, DF), lambda i: (0, 0)),
            pl.BlockSpec((1, DC), lambda i: (0, 0)),
        ],
        out_specs=[pl.BlockSpec((_BR, DC), lambda i: (i, 0))] * 2,
        out_shape=[jax.ShapeDtypeStruct((N, DC), jnp.float32)] * 2,
    )(ga, gb, wrel, wroot, b)


def _comb_body(a_ref, b_ref, o_ref):
    o_ref[...] = a_ref[...] + b_ref[...]


def _combine(pa, pb):
    return pl.pallas_call(
        _comb_body,
        grid=(N // _BR,),
        in_specs=[pl.BlockSpec((_BR, DC), lambda i: (i, 0))] * 2,
        out_specs=pl.BlockSpec((_BR, DC), lambda i: (i, 0)),
        out_shape=jax.ShapeDtypeStruct((N, DC), jnp.float32),
    )(pa, pb)


@functools.cache
def _mesh():
    return plsc.VectorSubcoreMesh(
        core_axis_name="c", subcore_axis_name="s", num_cores=NC, num_subcores=NS)


def _init_acc(s, z_hbm, acc, sem):
    def body(k, _):
        ci = k * NS + s
        @pl.when(ci < NCH)
        def _():
            r0 = ci * WB
            pltpu.async_copy(z_hbm.at[pl.ds(r0, WB)], acc.at[pl.ds(r0, WB)], sem)
        return 0
    lax.fori_loop(0, KWB, body, 0)
    def drain(k, _):
        ci = k * NS + s
        @pl.when(ci < NCH)
        def _():
            pltpu.make_async_copy(z_hbm.at[pl.ds(0, WB)], acc.at[pl.ds(0, WB)], sem).wait()
        return 0
    lax.fori_loop(0, KWB, drain, 0)


def _writeback(s, acc, out_hbm, sem):
    def body(k, _):
        ci = k * NS + s
        @pl.when(ci < NCH)
        def _():
            r0 = ci * WB
            pltpu.async_copy(acc.at[pl.ds(r0, WB)], out_hbm.at[pl.ds(r0, WB)], sem)
        return 0
    lax.fori_loop(0, KWB, body, 0)
    def drain(k, _):
        ci = k * NS + s
        @pl.when(ci < NCH)
        def _():
            pltpu.make_async_copy(acc.at[pl.ds(0, WB)], out_hbm.at[pl.ds(0, WB)], sem).wait()
        return 0
    lax.fori_loop(0, KWB, drain, 0)


def _edge_loop(n_rows, y_hbm, src_v, dst_v, acc, b0, b1, gs0, gs1, ss0, ss1):
    def gather(i, buf, sem):
        pltpu.async_copy(y_hbm.at[src_v.at[i]], buf, sem)

    def scat(i, buf, sem):
        pltpu.async_copy(buf, acc.at[dst_v.at[i]], sem, add=True)

    def wait_g(buf, sem):
        pltpu.make_async_copy(y_hbm.at[src_v.at[0]], buf, sem).wait()

    def wait_s(buf, sem):
        pltpu.make_async_copy(buf, acc.at[dst_v.at[0]], sem).wait()

    gather(0, b0, gs0)

    def body(k, _):
        i0 = 2 * k
        pl.when(k > 0)(lambda: wait_s(b1, ss1))
        gather(i0 + 1, b1, gs1)
        wait_g(b0, gs0)
        scat(i0, b0, ss0)

        @pl.when(i0 + 2 < n_rows)
        def _():
            wait_s(b0, ss0)
            gather(i0 + 2, b0, gs0)

        wait_g(b1, gs1)
        scat(i0 + 1, b1, ss1)
        return 0

    lax.fori_loop(0, n_rows // 2, body, 0)
    wait_s(b0, ss0)
    wait_s(b1, ss1)


@functools.cache
def _sc_layer1():
    @functools.partial(
        pl.kernel,
        out_type=(jax.ShapeDtypeStruct((N, DC), jnp.float32),) * 2,
        mesh=_mesh(),
        scratch_types=[
            pltpu.VMEM((PHROWS, CHUNK), jnp.int32),
            pltpu.VMEM((PHROWS, CHUNK), jnp.int32),
            pltpu.VMEM((CHUNK, DC), jnp.float32),
            pltpu.VMEM((CHUNK, DC), jnp.float32),
            pltpu.VMEM_SHARED((N, DC), jnp.float32),
            pltpu.SemaphoreType.DMA, pltpu.SemaphoreType.DMA,
            pltpu.SemaphoreType.DMA, pltpu.SemaphoreType.DMA,
        ],
    )
    def sc1(src_hbm, dst_hbm, ya_hbm, yb_hbm, za_hbm, zb_hbm,
            ga_hbm, gb_hbm, src_v, dst_v, b0, b1, acc, gs0, gs1, ss0, ss1):
        c = lax.axis_index("c")
        s = lax.axis_index("s")

        def run(y_hbm, z_hbm, out_hbm):
            _init_acc(s, z_hbm, acc, gs0)
            plsc.subcore_barrier()
            for p in range(PH):
                r0 = s * ROWS_L1 + p * PHROWS
                pltpu.sync_copy(src_hbm.at[pl.ds(r0, PHROWS)], src_v)
                pltpu.sync_copy(dst_hbm.at[pl.ds(r0, PHROWS)], dst_v)
                _edge_loop(PHROWS, y_hbm, src_v, dst_v, acc, b0, b1, gs0, gs1, ss0, ss1)
            plsc.subcore_barrier()
            _writeback(s, acc, out_hbm, gs0)

        pl.when(c == 0)(lambda: run(ya_hbm, za_hbm, ga_hbm))
        pl.when(c == 1)(lambda: run(yb_hbm, zb_hbm, gb_hbm))

    return sc1


@functools.cache
def _sc_layer2():
    @functools.partial(
        pl.kernel,
        out_type=(jax.ShapeDtypeStruct((N, DC), jnp.float32),) * 2,
        mesh=_mesh(),
        scratch_types=[
            pltpu.VMEM((ROWS_L2, CHUNK), jnp.int32),
            pltpu.VMEM((ROWS_L2, CHUNK), jnp.int32),
            pltpu.VMEM((CHUNK, DC), jnp.float32),
            pltpu.VMEM((CHUNK, DC), jnp.float32),
            pltpu.VMEM_SHARED((N, DC), jnp.float32),
            pltpu.SemaphoreType.DMA, pltpu.SemaphoreType.DMA,
            pltpu.SemaphoreType.DMA, pltpu.SemaphoreType.DMA,
        ],
    )
    def sc2(src_hbm, dst_hbm, y_hbm, z_hbm,
            oa_hbm, ob_hbm, src_v, dst_v, b0, b1, acc, gs0, gs1, ss0, ss1):
        c = lax.axis_index("c")
        s = lax.axis_index("s")
        r0 = (c * NS + s) * ROWS_L2
        pltpu.sync_copy(src_hbm.at[pl.ds(r0, ROWS_L2)], src_v)
        pltpu.sync_copy(dst_hbm.at[pl.ds(r0, ROWS_L2)], dst_v)
        _init_acc(s, z_hbm, acc, gs0)
        plsc.subcore_barrier()
        _edge_loop(ROWS_L2, y_hbm, src_v, dst_v, acc, b0, b1, gs0, gs1, ss0, ss1)
        plsc.subcore_barrier()
        pl.when(c == 0)(lambda: _writeback(s, acc, oa_hbm, gs0))
        pl.when(c == 1)(lambda: _writeback(s, acc, ob_hbm, gs0))

    return sc2


def kernel(edge_index, x, W1_rel, W1_root, b1, W2_rel, W2_root, b2):
    src2d = edge_index[0].reshape(ROWS, CHUNK)
    dst2d = edge_index[1].reshape(ROWS, CHUNK)
    y1a, y1b, z1a, z1b = _linear1(x, W1_rel, W1_root, b1.reshape(1, -1))
    g1a, g1b = _sc_layer1()(src2d, dst2d, y1a, y1b, z1a, z1b)
    y2, z2h = _linear2(g1a, g1b, W2_rel, W2_root, b2.reshape(1, -1))
    pa, pb = _sc_layer2()(src2d, dst2d, y2, z2h)
    return _combine(pa, pb)
